# baseline (speedup 1.0000x reference)
import functools

import jax
import jax.numpy as jnp
from jax import lax
from jax.experimental import pallas as pl
from jax.experimental.pallas import tpu as pltpu
from jax.experimental.pallas import tpu_sc as plsc

NC = 2
NS = 16
L = 16
NW = NC * NS
DP = 112
CHUNK = 80

_SC_PARAMS = pltpu.CompilerParams(use_tc_tiling_on_sc=False)


NB = 6


def _agg_body(xcp, src3, dst3, z0, acc_out, *rest):
    rows = rest[0:NB]
    src_r = rest[NB:2 * NB]
    dst_r = rest[2 * NB:3 * NB]
    gsem = rest[3 * NB:4 * NB]
    ssem = rest[4 * NB:5 * NB]
    isrc = rest[5 * NB:6 * NB]
    idst = rest[6 * NB:7 * NB]
    acc_sh = rest[7 * NB]
    cid = lax.axis_index("c")
    sid = lax.axis_index("s")
    wid = sid * NC + cid
    n_tile = z0.shape[0]
    nchunk = src3.shape[1]
    ngrp = nchunk // NB
    rem = nchunk % NB

    pltpu.sync_copy(z0, acc_sh.at[pl.ds(sid * n_tile, n_tile)])
    plsc.subcore_barrier()

    for r in range(rem):
        j = ngrp * NB + r
        pltpu.sync_copy(src3.at[wid, j], src_r[0])
        pltpu.sync_copy(dst3.at[wid, j], dst_r[0])
        pltpu.sync_copy(xcp.at[src_r[0]], rows[0])
        pltpu.sync_copy(rows[0], acc_sh.at[dst_r[0]], add=True)

    def wait_bytes(dst_ref, sem):
        dummy = xcp.at[pl.ds(0, CHUNK)] if dst_ref.ndim == 2 else src3.at[wid, 0]
        pltpu.make_async_copy(dummy, dst_ref, sem).wait()

    for b in range(NB):
        pltpu.async_copy(src3.at[wid, b], src_r[b], isrc[b])
        pltpu.async_copy(dst3.at[wid, b], dst_r[b], idst[b])

    def body(g, carry):
        base = g * NB
        for b in range(NB):
            wait_bytes(src_r[b], isrc[b])
            wait_bytes(dst_r[b], idst[b])
            pltpu.async_copy(xcp.at[src_r[b]], rows[b], gsem[b])
        for b in range(NB):
            wait_bytes(rows[b], gsem[b])
            nxt = jnp.minimum(base + NB + b, nchunk - 1)
            pltpu.async_copy(src3.at[wid, nxt], src_r[b], isrc[b])
            pltpu.async_copy(rows[b], acc_sh.at[dst_r[b]], ssem[b], add=True)
        for b in range(NB):
            wait_bytes(rows[b], ssem[b])
            nxt = jnp.minimum(base + NB + b, nchunk - 1)
            pltpu.async_copy(dst3.at[wid, nxt], dst_r[b], idst[b])
        return carry

    lax.fori_loop(0, ngrp, body, 0)
    for b in range(NB):
        wait_bytes(src_r[b], isrc[b])
        wait_bytes(dst_r[b], idst[b])
    plsc.subcore_barrier()
    pltpu.sync_copy(acc_sh.at[pl.ds(sid * n_tile, n_tile)],
                    acc_out.at[cid, pl.ds(sid * n_tile, n_tile)])


def _mlp_body(xcp_ref, a0_ref, a1_ref, w1_ref, b1_ref, w2_ref, b2_ref,
              wl_ref, bl_ref, s_ref):
    z = xcp_ref[...] + a0_ref[0] + a1_ref[0]
    h = jnp.dot(z, w1_ref[...], preferred_element_type=jnp.float32) + b1_ref[...]
    h = jnp.maximum(h, 0.0)
    h = jnp.dot(h, w2_ref[...], preferred_element_type=jnp.float32) + b2_ref[...]
    s_ref[...] = (jnp.dot(h, wl_ref[...], preferred_element_type=jnp.float32)
                  + bl_ref[0, 0])


_W = 16


def _edge_gather_body(s_hbm, dst3, out_hbm, dst_v, out_v, sem, s_sh):
    cid = lax.axis_index("c")
    sid = lax.axis_index("s")
    wid = sid * NC + cid
    nchunk = dst_v.shape[0]

    @pl.when(sid == 0)
    def _():
        pltpu.sync_copy(s_hbm, s_sh)

    pltpu.sync_copy(dst3.at[wid], dst_v)
    plsc.subcore_barrier()

    def wait_one():
        pltpu.make_async_copy(s_hbm.at[pl.ds(0, CHUNK)], out_v.at[0],
                              sem).wait()

    def body(j, carry):
        pltpu.async_copy(s_sh.at[dst_v.at[j]], out_v.at[j], sem)

        @pl.when(j >= _W)
        def _():
            wait_one()

        return carry

    lax.fori_loop(0, nchunk, body, 0)
    for _ in range(_W):
        wait_one()
    pltpu.sync_copy(out_v, out_hbm.at[wid])


def kernel(x, edge_index, pos_embeddings, W1, b1, W2, b2,
           W1p, b1p, W2p, b2p, Wl, bl):
    _, N, F = x.shape
    P = pos_embeddings.shape[2]
    E = edge_index.shape[1]
    H = W1.shape[1]
    eb = E // NW
    nchunk = eb // CHUNK
    n_tile = N // NS

    xf = x.reshape(N, F)
    pf = pos_embeddings.reshape(N, P)
    xcp = jnp.concatenate(
        [xf, pf, jnp.zeros((N, DP - F - P), jnp.float32)], axis=1)
    w1p = jnp.concatenate(
        [W1, jnp.zeros((DP - F - P, H), W1.dtype)], axis=0)
    src3 = edge_index[0].reshape(NW, nchunk, CHUNK)
    dst3 = edge_index[1].reshape(NW, nchunk, CHUNK)
    z0 = jnp.zeros((n_tile, DP), jnp.float32)

    mesh = plsc.VectorSubcoreMesh(core_axis_name="c", subcore_axis_name="s")

    agg_call = functools.partial(
        pl.kernel,
        out_type=jax.ShapeDtypeStruct((NC, N, DP), jnp.float32),
        mesh=mesh,
        compiler_params=_SC_PARAMS,
        scratch_types=(
            [pltpu.VMEM((CHUNK, DP), jnp.float32)] * NB
            + [pltpu.VMEM((CHUNK,), jnp.int32)] * (2 * NB)
            + [pltpu.SemaphoreType.DMA] * (4 * NB)
            + [pltpu.VMEM_SHARED((N, DP), jnp.float32)]
        ),
    )(_agg_body)
    acc = agg_call(xcp, src3, dst3, z0)

    BN = 2000
    s2 = pl.pallas_call(
        _mlp_body,
        out_shape=jax.ShapeDtypeStruct((N, 1), jnp.float32),
        grid=(N // BN,),
        in_specs=[
            pl.BlockSpec((BN, DP), lambda i: (i, 0)),
            pl.BlockSpec((1, BN, DP), lambda i: (0, i, 0)),
            pl.BlockSpec((1, BN, DP), lambda i: (1, i, 0)),
            pl.BlockSpec((DP, H), lambda i: (0, 0)),
            pl.BlockSpec((1, H), lambda i: (0, 0)),
            pl.BlockSpec((H, H), lambda i: (0, 0)),
            pl.BlockSpec((1, H), lambda i: (0, 0)),
            pl.BlockSpec((H, 1), lambda i: (0, 0)),
            pl.BlockSpec((1, 1), lambda i: (0, 0)),
        ],
        out_specs=pl.BlockSpec((BN, 1), lambda i: (i, 0)),
    )(xcp, acc, acc, w1p, b1.reshape(1, H), W2, b2.reshape(1, H),
      Wl, bl.reshape(1, 1))

    gather_call = functools.partial(
        pl.kernel,
        out_type=jax.ShapeDtypeStruct((NW, nchunk, CHUNK), jnp.float32),
        mesh=mesh,
        compiler_params=_SC_PARAMS,
        scratch_types=[
            pltpu.VMEM((nchunk, CHUNK), jnp.int32),
            pltpu.VMEM((nchunk, CHUNK), jnp.float32),
            pltpu.SemaphoreType.DMA,
            pltpu.VMEM_SHARED((N,), jnp.float32),
        ],
    )(_edge_gather_body)
    out3 = gather_call(s2.reshape(N), dst3)
    return out3.reshape(E)


# speedup vs baseline: 1.0095x; 1.0095x over previous
import functools

import jax
import jax.numpy as jnp
from jax import lax
from jax.experimental import pallas as pl
from jax.experimental.pallas import tpu as pltpu
from jax.experimental.pallas import tpu_sc as plsc

NC = 2
NS = 16
L = 16
NW = NC * NS
DP = 112
CHUNK = 80

_SC_PARAMS = pltpu.CompilerParams(use_tc_tiling_on_sc=False)


NB = 5


def _agg_body(xcp, src3, dst3, z0, acc_out, *rest):
    rows = rest[0:NB]
    src_r = rest[NB:2 * NB]
    dst_r = rest[2 * NB:3 * NB]
    gsem = rest[3 * NB:4 * NB]
    ssem = rest[4 * NB:5 * NB]
    isrc = rest[5 * NB:6 * NB]
    idst = rest[6 * NB:7 * NB]
    acc_sh = rest[7 * NB]
    cid = lax.axis_index("c")
    sid = lax.axis_index("s")
    wid = sid * NC + cid
    n_tile = z0.shape[0]
    nchunk = src3.shape[1]
    ngrp = nchunk // NB
    rem = nchunk % NB

    pltpu.sync_copy(z0, acc_sh.at[pl.ds(sid * n_tile, n_tile)])
    plsc.subcore_barrier()

    for r in range(rem):
        j = ngrp * NB + r
        pltpu.sync_copy(src3.at[wid, j], src_r[0])
        pltpu.sync_copy(dst3.at[wid, j], dst_r[0])
        pltpu.sync_copy(xcp.at[src_r[0]], rows[0])
        pltpu.sync_copy(rows[0], acc_sh.at[dst_r[0]], add=True)

    def wait_bytes(dst_ref, sem):
        dummy = xcp.at[pl.ds(0, CHUNK)] if dst_ref.ndim == 2 else src3.at[wid, 0]
        pltpu.make_async_copy(dummy, dst_ref, sem).wait()

    for b in range(NB):
        pltpu.async_copy(src3.at[wid, b], src_r[b], isrc[b])
        pltpu.async_copy(dst3.at[wid, b], dst_r[b], idst[b])

    def body(g, carry):
        base = g * NB
        for b in range(NB):
            wait_bytes(src_r[b], isrc[b])
            wait_bytes(dst_r[b], idst[b])
            pltpu.async_copy(xcp.at[src_r[b]], rows[b], gsem[b])
        for b in range(NB):
            wait_bytes(rows[b], gsem[b])
            nxt = jnp.minimum(base + NB + b, nchunk - 1)
            pltpu.async_copy(src3.at[wid, nxt], src_r[b], isrc[b])
            pltpu.async_copy(rows[b], acc_sh.at[dst_r[b]], ssem[b], add=True)
        for b in range(NB):
            wait_bytes(rows[b], ssem[b])
            nxt = jnp.minimum(base + NB + b, nchunk - 1)
            pltpu.async_copy(dst3.at[wid, nxt], dst_r[b], idst[b])
        return carry

    lax.fori_loop(0, ngrp, body, 0)
    for b in range(NB):
        wait_bytes(src_r[b], isrc[b])
        wait_bytes(dst_r[b], idst[b])
    plsc.subcore_barrier()
    pltpu.sync_copy(acc_sh.at[pl.ds(sid * n_tile, n_tile)],
                    acc_out.at[cid, pl.ds(sid * n_tile, n_tile)])


def _mlp_body(xcp_ref, a0_ref, a1_ref, w1_ref, b1_ref, w2_ref, b2_ref,
              wl_ref, bl_ref, s_ref):
    v = jnp.dot(w2_ref[...], wl_ref[...], preferred_element_type=jnp.float32)
    c = jnp.dot(b2_ref[...], wl_ref[...], preferred_element_type=jnp.float32)
    z = xcp_ref[...] + a0_ref[0] + a1_ref[0]
    h = jnp.dot(z, w1_ref[...], preferred_element_type=jnp.float32) + b1_ref[...]
    h = jnp.maximum(h, 0.0)
    s_ref[...] = (jnp.dot(h, v, preferred_element_type=jnp.float32)
                  + c[0, 0] + bl_ref[0, 0])


_W = 32


def _edge_gather_body(s_hbm, dst3, out_hbm, dst_v, out_v, sem, s_sh):
    cid = lax.axis_index("c")
    sid = lax.axis_index("s")
    wid = sid * NC + cid
    nchunk = dst_v.shape[0]

    @pl.when(sid == 0)
    def _():
        pltpu.sync_copy(s_hbm, s_sh)

    pltpu.sync_copy(dst3.at[wid], dst_v)
    plsc.subcore_barrier()

    def wait_one():
        pltpu.make_async_copy(s_hbm.at[pl.ds(0, CHUNK)], out_v.at[0],
                              sem).wait()

    def body(j, carry):
        pltpu.async_copy(s_sh.at[dst_v.at[j]], out_v.at[j], sem)

        @pl.when(j >= _W)
        def _():
            wait_one()

        return carry

    lax.fori_loop(0, nchunk, body, 0)
    for _ in range(_W):
        wait_one()
    pltpu.sync_copy(out_v, out_hbm.at[wid])


def kernel(x, edge_index, pos_embeddings, W1, b1, W2, b2,
           W1p, b1p, W2p, b2p, Wl, bl):
    _, N, F = x.shape
    P = pos_embeddings.shape[2]
    E = edge_index.shape[1]
    H = W1.shape[1]
    eb = E // NW
    nchunk = eb // CHUNK
    n_tile = N // NS

    xf = x.reshape(N, F)
    pf = pos_embeddings.reshape(N, P)
    xcp = jnp.concatenate(
        [xf, pf, jnp.zeros((N, DP - F - P), jnp.float32)], axis=1)
    w1p = jnp.concatenate(
        [W1, jnp.zeros((DP - F - P, H), W1.dtype)], axis=0)
    src3 = edge_index[0].reshape(NW, nchunk, CHUNK)
    dst3 = edge_index[1].reshape(NW, nchunk, CHUNK)
    z0 = jnp.zeros((n_tile, DP), jnp.float32)

    mesh = plsc.VectorSubcoreMesh(core_axis_name="c", subcore_axis_name="s")

    agg_call = functools.partial(
        pl.kernel,
        out_type=jax.ShapeDtypeStruct((NC, N, DP), jnp.float32),
        mesh=mesh,
        compiler_params=_SC_PARAMS,
        scratch_types=(
            [pltpu.VMEM((CHUNK, DP), jnp.float32)] * NB
            + [pltpu.VMEM((CHUNK,), jnp.int32)] * (2 * NB)
            + [pltpu.SemaphoreType.DMA] * (4 * NB)
            + [pltpu.VMEM_SHARED((N, DP), jnp.float32)]
        ),
    )(_agg_body)
    acc = agg_call(xcp, src3, dst3, z0)

    BN = 2000
    s2 = pl.pallas_call(
        _mlp_body,
        out_shape=jax.ShapeDtypeStruct((N, 1), jnp.float32),
        grid=(N // BN,),
        in_specs=[
            pl.BlockSpec((BN, DP), lambda i: (i, 0)),
            pl.BlockSpec((1, BN, DP), lambda i: (0, i, 0)),
            pl.BlockSpec((1, BN, DP), lambda i: (1, i, 0)),
            pl.BlockSpec((DP, H), lambda i: (0, 0)),
            pl.BlockSpec((1, H), lambda i: (0, 0)),
            pl.BlockSpec((H, H), lambda i: (0, 0)),
            pl.BlockSpec((1, H), lambda i: (0, 0)),
            pl.BlockSpec((H, 1), lambda i: (0, 0)),
            pl.BlockSpec((1, 1), lambda i: (0, 0)),
        ],
        out_specs=pl.BlockSpec((BN, 1), lambda i: (i, 0)),
    )(xcp, acc, acc, w1p, b1.reshape(1, H), W2, b2.reshape(1, H),
      Wl, bl.reshape(1, 1))

    gather_call = functools.partial(
        pl.kernel,
        out_type=jax.ShapeDtypeStruct((NW, nchunk, CHUNK), jnp.float32),
        mesh=mesh,
        compiler_params=_SC_PARAMS,
        scratch_types=[
            pltpu.VMEM((nchunk, CHUNK), jnp.int32),
            pltpu.VMEM((nchunk, CHUNK), jnp.float32),
            pltpu.SemaphoreType.DMA,
            pltpu.VMEM_SHARED((N,), jnp.float32),
        ],
    )(_edge_gather_body)
    out3 = gather_call(s2.reshape(N), dst3)
    return out3.reshape(E)


# speedup vs baseline: 1.0279x; 1.0181x over previous
import functools

import jax
import jax.numpy as jnp
from jax import lax
from jax.experimental import pallas as pl
from jax.experimental.pallas import tpu as pltpu
from jax.experimental.pallas import tpu_sc as plsc

NC = 2
NS = 16
L = 16
NW = NC * NS
DP = 112
CHUNK = 80

_SC_PARAMS = pltpu.CompilerParams(use_tc_tiling_on_sc=False)


NB = 5


def _agg_body(xcp, src3, dst3, acc_out, *rest):
    rows = rest[0:NB]
    src_r = rest[NB:2 * NB]
    dst_r = rest[2 * NB:3 * NB]
    gsem = rest[3 * NB:4 * NB]
    ssem = rest[4 * NB:5 * NB]
    isrc = rest[5 * NB:6 * NB]
    idst = rest[6 * NB:7 * NB]
    acc_sh = rest[7 * NB]
    cid = lax.axis_index("c")
    sid = lax.axis_index("s")
    wid = sid * NC + cid
    n_tile = acc_sh.shape[0] // NS
    nchunk = src3.shape[1]
    ngrp = nchunk // NB
    rem = nchunk % NB

    def zfill(r, carry):
        for k in range(DP // L):
            rows[0][r, pl.ds(k * L, L)] = jnp.zeros((L,), jnp.float32)
        return carry

    lax.fori_loop(0, CHUNK, zfill, 0)
    nfull = n_tile // CHUNK
    tail = n_tile % CHUNK
    for i in range(nfull):
        pltpu.async_copy(
            rows[0], acc_sh.at[pl.ds(sid * n_tile + i * CHUNK, CHUNK)],
            ssem[0])
    if tail:
        pltpu.async_copy(
            rows[0].at[pl.ds(0, tail)],
            acc_sh.at[pl.ds(sid * n_tile + nfull * CHUNK, tail)], ssem[0])
    for i in range(nfull):
        pltpu.make_async_copy(rows[0], acc_sh.at[pl.ds(0, CHUNK)],
                              ssem[0]).wait()
    if tail:
        pltpu.make_async_copy(rows[0].at[pl.ds(0, tail)],
                              acc_sh.at[pl.ds(0, tail)], ssem[0]).wait()
    plsc.subcore_barrier()

    for r in range(rem):
        j = ngrp * NB + r
        pltpu.sync_copy(src3.at[wid, j], src_r[0])
        pltpu.sync_copy(dst3.at[wid, j], dst_r[0])
        pltpu.sync_copy(xcp.at[src_r[0]], rows[0])
        pltpu.sync_copy(rows[0], acc_sh.at[dst_r[0]], add=True)

    def wait_bytes(dst_ref, sem):
        dummy = xcp.at[pl.ds(0, CHUNK)] if dst_ref.ndim == 2 else src3.at[wid, 0]
        pltpu.make_async_copy(dummy, dst_ref, sem).wait()

    for b in range(NB):
        pltpu.async_copy(src3.at[wid, b], src_r[b], isrc[b])
        pltpu.async_copy(dst3.at[wid, b], dst_r[b], idst[b])

    def body(g, carry):
        base = g * NB
        for b in range(NB):
            wait_bytes(src_r[b], isrc[b])
            wait_bytes(dst_r[b], idst[b])
            pltpu.async_copy(xcp.at[src_r[b]], rows[b], gsem[b])
        for b in range(NB):
            wait_bytes(rows[b], gsem[b])
            nxt = jnp.minimum(base + NB + b, nchunk - 1)
            pltpu.async_copy(src3.at[wid, nxt], src_r[b], isrc[b])
            pltpu.async_copy(rows[b], acc_sh.at[dst_r[b]], ssem[b], add=True)
        for b in range(NB):
            wait_bytes(rows[b], ssem[b])
            nxt = jnp.minimum(base + NB + b, nchunk - 1)
            pltpu.async_copy(dst3.at[wid, nxt], dst_r[b], idst[b])
        return carry

    lax.fori_loop(0, ngrp, body, 0)
    for b in range(NB):
        wait_bytes(src_r[b], isrc[b])
        wait_bytes(dst_r[b], idst[b])
    plsc.subcore_barrier()
    pltpu.sync_copy(acc_sh.at[pl.ds(sid * n_tile, n_tile)],
                    acc_out.at[cid, pl.ds(sid * n_tile, n_tile)])


def _mlp_body(xcp_ref, a0_ref, a1_ref, w1_ref, b1_ref, w2_ref, b2_ref,
              wl_ref, bl_ref, s_ref):
    z = xcp_ref[...] + a0_ref[0] + a1_ref[0]
    h = jnp.dot(z, w1_ref[...], preferred_element_type=jnp.float32) + b1_ref[...]
    h = jnp.maximum(h, 0.0)
    h = jnp.dot(h, w2_ref[...], preferred_element_type=jnp.float32) + b2_ref[...]
    s_ref[...] = (jnp.dot(h, wl_ref[...], preferred_element_type=jnp.float32)
                  + bl_ref[0, 0])


_W = 32


def _edge_gather_body(s_hbm, dst3, out_hbm, dst_v, out_v, sem, s_sh):
    cid = lax.axis_index("c")
    sid = lax.axis_index("s")
    wid = sid * NC + cid
    nchunk = dst_v.shape[0]

    @pl.when(sid == 0)
    def _():
        pltpu.sync_copy(s_hbm, s_sh)

    pltpu.sync_copy(dst3.at[wid], dst_v)
    plsc.subcore_barrier()

    def wait_one():
        pltpu.make_async_copy(s_hbm.at[pl.ds(0, CHUNK)], out_v.at[0],
                              sem).wait()

    def body(j, carry):
        pltpu.async_copy(s_sh.at[dst_v.at[j]], out_v.at[j], sem)

        @pl.when(j >= _W)
        def _():
            wait_one()

        return carry

    lax.fori_loop(0, nchunk, body, 0)
    for _ in range(_W):
        wait_one()
    pltpu.sync_copy(out_v, out_hbm.at[wid])


def kernel(x, edge_index, pos_embeddings, W1, b1, W2, b2,
           W1p, b1p, W2p, b2p, Wl, bl):
    _, N, F = x.shape
    P = pos_embeddings.shape[2]
    E = edge_index.shape[1]
    H = W1.shape[1]
    eb = E // NW
    nchunk = eb // CHUNK
    n_tile = N // NS

    xf = x.reshape(N, F)
    pf = pos_embeddings.reshape(N, P)
    xcp = jnp.concatenate(
        [xf, pf, jnp.zeros((N, DP - F - P), jnp.float32)], axis=1)
    w1p = jnp.concatenate(
        [W1, jnp.zeros((DP - F - P, H), W1.dtype)], axis=0)
    src3 = edge_index[0].reshape(NW, nchunk, CHUNK)
    dst3 = edge_index[1].reshape(NW, nchunk, CHUNK)

    mesh = plsc.VectorSubcoreMesh(core_axis_name="c", subcore_axis_name="s")

    agg_call = functools.partial(
        pl.kernel,
        out_type=jax.ShapeDtypeStruct((NC, N, DP), jnp.float32),
        mesh=mesh,
        compiler_params=_SC_PARAMS,
        scratch_types=(
            [pltpu.VMEM((CHUNK, DP), jnp.float32)] * NB
            + [pltpu.VMEM((CHUNK,), jnp.int32)] * (2 * NB)
            + [pltpu.SemaphoreType.DMA] * (4 * NB)
            + [pltpu.VMEM_SHARED((N, DP), jnp.float32)]
        ),
    )(_agg_body)
    acc = agg_call(xcp, src3, dst3)

    BN = 2000
    s2 = pl.pallas_call(
        _mlp_body,
        out_shape=jax.ShapeDtypeStruct((N, 1), jnp.float32),
        grid=(N // BN,),
        in_specs=[
            pl.BlockSpec((BN, DP), lambda i: (i, 0)),
            pl.BlockSpec((1, BN, DP), lambda i: (0, i, 0)),
            pl.BlockSpec((1, BN, DP), lambda i: (1, i, 0)),
            pl.BlockSpec((DP, H), lambda i: (0, 0)),
            pl.BlockSpec((1, H), lambda i: (0, 0)),
            pl.BlockSpec((H, H), lambda i: (0, 0)),
            pl.BlockSpec((1, H), lambda i: (0, 0)),
            pl.BlockSpec((H, 1), lambda i: (0, 0)),
            pl.BlockSpec((1, 1), lambda i: (0, 0)),
        ],
        out_specs=pl.BlockSpec((BN, 1), lambda i: (i, 0)),
    )(xcp, acc, acc, w1p, b1.reshape(1, H), W2, b2.reshape(1, H),
      Wl, bl.reshape(1, 1))

    gather_call = functools.partial(
        pl.kernel,
        out_type=jax.ShapeDtypeStruct((NW, nchunk, CHUNK), jnp.float32),
        mesh=mesh,
        compiler_params=_SC_PARAMS,
        scratch_types=[
            pltpu.VMEM((nchunk, CHUNK), jnp.int32),
            pltpu.VMEM((nchunk, CHUNK), jnp.float32),
            pltpu.SemaphoreType.DMA,
            pltpu.VMEM_SHARED((N,), jnp.float32),
        ],
    )(_edge_gather_body)
    out3 = gather_call(s2.reshape(N), dst3)
    return out3.reshape(E)
